# baseline (speedup 1.0000x reference)
import functools

import jax
import jax.numpy as jnp
import numpy as np
from jax import lax
from jax.experimental import pallas as pl
from jax.experimental.pallas import tpu as pltpu
from jax.experimental.pallas import tpu_sc as plsc

N = 2097152
NC, NS, L = 2, 16, 16
NW = NC * NS
CHUNK = N // NW
BLK = 8192
NBLK = CHUNK // BLK
NBINS = 2048
MSB = np.int32(-2147483648)

_mesh = plsc.VectorSubcoreMesh(
    core_axis_name="c", subcore_axis_name="s", num_cores=NC, num_subcores=NS
)


def _wid():
    return lax.axis_index("s") * NC + lax.axis_index("c")


def _lane_iota():
    return lax.iota(jnp.int32, 16)


def _zero_hist(hist_v):
    z = jnp.zeros((16,), jnp.int32)

    def body(i, _):
        for u in range(16):
            hist_v[pl.ds(i * 256 + u * 16, 16)] = z
        return 0

    lax.fori_loop(0, NBINS * 16 // 256, body, 0)


def _reduce_hist(hist_v, red_v, nbins):
    def body(j, _):
        acc = hist_v[pl.ds(j * 16, 16)]
        for l in range(1, 16):
            acc = acc + hist_v[pl.ds(l * NBINS + j * 16, 16)]
        red_v[pl.ds(j * 16, 16)] = acc
        return 0

    lax.fori_loop(0, nbins // 16, body, 0)


def _pass1_body(pred_hbm, hist_hbm, mkey_hbm, buf_f, buf_i, hist_v, red_v):
    wid = _wid()
    base = wid * CHUNK
    lane = _lane_iota()
    laneoff = lane * NBINS
    ones = jnp.ones((16,), jnp.int32)

    _zero_hist(hist_v)

    def blk_body(b, _):
        off = base + b * BLK
        pltpu.sync_copy(pred_hbm.at[pl.ds(off, BLK)], buf_f)

        def grp_body(i, _):
            gidx = lane * 8 + i * 128
            vs = [plsc.load_gather(buf_f, [gidx + j]) for j in range(8)]
            m = vs[0]
            amax = jnp.zeros((16,), jnp.int32)
            for j in range(1, 8):
                gt = vs[j] > m
                m = jnp.where(gt, vs[j], m)
                amax = jnp.where(gt, jnp.int32(j), amax)
            for j in range(8):
                s = plsc.bitcast(vs[j], jnp.int32)
                s = jnp.where(s == MSB, jnp.int32(0), s)
                ukey = s ^ ((s >> 31) | MSB)
                mk = jnp.where(amax == j, jnp.int32(-1), ukey)
                plsc.store_scatter(buf_i, [gidx + j], mk)
                bin_ = lax.shift_right_logical(mk, 21)
                plsc.addupdate_scatter(hist_v, [bin_ + laneoff], ones)
            return 0

        plsc.parallel_loop(0, BLK // 128, 1, unroll=4)(
            lambda i: grp_body(i, 0) and None)
        pltpu.sync_copy(buf_i, mkey_hbm.at[pl.ds(off, BLK)])
        return 0

    lax.fori_loop(0, NBLK, blk_body, 0)

    _reduce_hist(hist_v, red_v, NBINS)
    pltpu.sync_copy(red_v, hist_hbm.at[pl.ds(wid * NBINS, NBINS)])


def _hist_pass_body(match_shift, bin_shift, bin_mask, nbins,
                    mkey_hbm, pfx_hbm, hist_hbm, buf_i, pfx_v, hist_v, red_v):
    wid = _wid()
    base = wid * CHUNK
    laneoff = _lane_iota() * NBINS
    ones = jnp.ones((16,), jnp.int32)

    pltpu.sync_copy(pfx_hbm, pfx_v)
    pfx = pfx_v[...]
    _zero_hist(hist_v)

    def blk_body(b, _):
        off = base + b * BLK
        pltpu.sync_copy(mkey_hbm.at[pl.ds(off, BLK)], buf_i)

        def vec_body(i, _):
            vals = []
            matches = []
            for u in range(8):
                v = buf_i[pl.ds(i * 128 + u * 16, 16)]
                vals.append(v)
                matches.append(lax.shift_right_logical(v, match_shift) == pfx)
            anym = matches[0]
            for u in range(1, 8):
                anym = anym | matches[u]

            @pl.when(jnp.any(anym))
            def _():
                for u in range(8):
                    bin_ = lax.shift_right_logical(vals[u], bin_shift) & bin_mask
                    plsc.addupdate_scatter(hist_v, [bin_ + laneoff], ones,
                                           mask=matches[u])

            return 0

        plsc.parallel_loop(0, BLK // 128, 1, unroll=4)(
            lambda i: vec_body(i, 0) and None)
        return 0

    lax.fori_loop(0, NBLK, blk_body, 0)

    _reduce_hist(hist_v, red_v, nbins)
    pltpu.sync_copy(red_v.at[pl.ds(0, nbins)],
                    hist_hbm.at[pl.ds(wid * nbins, nbins)])


def _apply_body(pred_hbm, mkey_hbm, thr_hbm, out_hbm, buf_f, buf_i, thr_v):
    wid = _wid()
    base = wid * CHUNK
    zero = jnp.zeros((16,), jnp.float32)

    pltpu.sync_copy(thr_hbm, thr_v)
    thr = thr_v[...]

    def blk_body(b, _):
        off = base + b * BLK
        pltpu.sync_copy(pred_hbm.at[pl.ds(off, BLK)], buf_f)
        pltpu.sync_copy(mkey_hbm.at[pl.ds(off, BLK)], buf_i)

        def vec_body(i, _):
            for u in range(8):
                sl = pl.ds(i * 128 + u * 16, 16)
                mk = buf_i[sl]
                keep = (mk ^ MSB) > thr
                buf_f[sl] = jnp.where(keep, buf_f[sl], zero)
            return 0

        plsc.parallel_loop(0, BLK // 128, 1, unroll=4)(
            lambda i: vec_body(i, 0) and None)
        pltpu.sync_copy(buf_f, out_hbm.at[pl.ds(off, BLK)])
        return 0

    lax.fori_loop(0, NBLK, blk_body, 0)


_pass1 = pl.kernel(
    _pass1_body,
    out_type=(
        jax.ShapeDtypeStruct((NW * NBINS,), jnp.int32),
        jax.ShapeDtypeStruct((N,), jnp.int32),
    ),
    mesh=_mesh,
    compiler_params=pltpu.CompilerParams(needs_layout_passes=False),
    scratch_types=[
        pltpu.VMEM((BLK,), jnp.float32),
        pltpu.VMEM((BLK,), jnp.int32),
        pltpu.VMEM((16 * NBINS,), jnp.int32),
        pltpu.VMEM((NBINS,), jnp.int32),
    ],
)

_pass2 = pl.kernel(
    functools.partial(_hist_pass_body, 21, 10, np.int32(0x7FF), 2048),
    out_type=jax.ShapeDtypeStruct((NW * 2048,), jnp.int32),
    mesh=_mesh,
    compiler_params=pltpu.CompilerParams(needs_layout_passes=False),
    scratch_types=[
        pltpu.VMEM((BLK,), jnp.int32),
        pltpu.VMEM((16,), jnp.int32),
        pltpu.VMEM((16 * NBINS,), jnp.int32),
        pltpu.VMEM((NBINS,), jnp.int32),
    ],
)

_pass3 = pl.kernel(
    functools.partial(_hist_pass_body, 10, 0, np.int32(0x3FF), 1024),
    out_type=jax.ShapeDtypeStruct((NW * 1024,), jnp.int32),
    mesh=_mesh,
    compiler_params=pltpu.CompilerParams(needs_layout_passes=False),
    scratch_types=[
        pltpu.VMEM((BLK,), jnp.int32),
        pltpu.VMEM((16,), jnp.int32),
        pltpu.VMEM((16 * NBINS,), jnp.int32),
        pltpu.VMEM((NBINS,), jnp.int32),
    ],
)

_apply = pl.kernel(
    _apply_body,
    out_type=jax.ShapeDtypeStruct((N,), jnp.float32),
    mesh=_mesh,
    compiler_params=pltpu.CompilerParams(needs_layout_passes=False),
    scratch_types=[
        pltpu.VMEM((BLK,), jnp.float32),
        pltpu.VMEM((BLK,), jnp.int32),
        pltpu.VMEM((16,), jnp.int32),
    ],
)


def _pick(hist_flat, nbins, r):
    g = jnp.sum(hist_flat.reshape(NW, nbins), axis=0)
    c = jnp.cumsum(g)
    b = jnp.argmax(c >= r).astype(jnp.int32)
    r_next = r - (c[b] - g[b])
    return b, r_next


def kernel(pred, points_num):
    r = jnp.int32(N) - jnp.asarray(points_num, jnp.int32)

    hist1, mkey = _pass1(pred)
    b1, r2 = _pick(hist1, 2048, r)

    hist2 = _pass2(mkey, jnp.full((16,), b1, jnp.int32))
    b2, r3 = _pick(hist2, 2048, r2)
    pfx2 = (b1 << 11) | b2

    hist3 = _pass3(mkey, jnp.full((16,), pfx2, jnp.int32))
    b3, _ = _pick(hist3, 1024, r3)
    thresh = (pfx2 << 10) | b3

    sthr = thresh ^ MSB
    return _apply(pred, mkey, jnp.full((16,), sthr, jnp.int32))


# speedup vs baseline: 1.8237x; 1.8237x over previous
import functools

import jax
import jax.numpy as jnp
import numpy as np
from jax import lax
from jax.experimental import pallas as pl
from jax.experimental.pallas import tpu as pltpu
from jax.experimental.pallas import tpu_sc as plsc

N = 2097152
NC, NS, L = 2, 16, 16
NW = NC * NS
CHUNK = N // NW
BLK = 8192
NBLK = CHUNK // BLK
NBINS = 2048
MSB = np.int32(-2147483648)

_mesh = plsc.VectorSubcoreMesh(
    core_axis_name="c", subcore_axis_name="s", num_cores=NC, num_subcores=NS
)


def _wid():
    return lax.axis_index("s") * NC + lax.axis_index("c")


def _lane_iota():
    return lax.iota(jnp.int32, 16)


def _zero_hist(hist_v):
    z = jnp.zeros((16,), jnp.int32)

    def body(i, _):
        for u in range(16):
            hist_v[pl.ds(i * 256 + u * 16, 16)] = z
        return 0

    lax.fori_loop(0, NBINS * 16 // 256, body, 0)


def _reduce_hist(hist_v, red_v, nbins):
    def body(j, _):
        acc = hist_v[pl.ds(j * 16, 16)]
        for l in range(1, 16):
            acc = acc + hist_v[pl.ds(l * NBINS + j * 16, 16)]
        red_v[pl.ds(j * 16, 16)] = acc
        return 0

    lax.fori_loop(0, nbins // 16, body, 0)


def _pass1_body(pred_hbm, hist_hbm, mkey_hbm,
                buf_f0, buf_f1, buf_i0, buf_i1, hist_v, red_v,
                sem_f0, sem_f1, sem_o0, sem_o1):
    wid = _wid()
    base = wid * CHUNK
    lane = _lane_iota()
    laneoff = lane * NBINS
    ones = jnp.ones((16,), jnp.int32)
    bufs_f = (buf_f0, buf_f1)
    bufs_i = (buf_i0, buf_i1)
    sems_f = (sem_f0, sem_f1)
    sems_o = (sem_o0, sem_o1)

    _zero_hist(hist_v)

    in_cp = [None, None]
    out_cp = [None, None]
    in_cp[0] = pltpu.async_copy(
        pred_hbm.at[pl.ds(base, BLK)], bufs_f[0], sems_f[0])

    for b in range(NBLK):
        p = b & 1
        buf_f, buf_i = bufs_f[p], bufs_i[p]
        in_cp[p].wait()
        if b + 1 < NBLK:
            q = (b + 1) & 1
            in_cp[q] = pltpu.async_copy(
                pred_hbm.at[pl.ds(base + (b + 1) * BLK, BLK)],
                bufs_f[q], sems_f[q])
        if out_cp[p] is not None:
            out_cp[p].wait()

        def grp_body(i):
            gidx = lane * 8 + i * 128
            vs = [plsc.load_gather(buf_f, [gidx + j]) for j in range(8)]
            m = vs[0]
            amax = jnp.zeros((16,), jnp.int32)
            for j in range(1, 8):
                gt = vs[j] > m
                m = jnp.where(gt, vs[j], m)
                amax = jnp.where(gt, jnp.int32(j), amax)
            for j in range(8):
                s = plsc.bitcast(vs[j], jnp.int32)
                s = jnp.where(s == MSB, jnp.int32(0), s)
                ukey = s ^ ((s >> 31) | MSB)
                mk = jnp.where(amax == j, jnp.int32(-1), ukey)
                plsc.store_scatter(buf_i, [gidx + j], mk)
                bin_ = lax.shift_right_logical(mk, 21)
                plsc.addupdate_scatter(hist_v, [bin_ + laneoff], ones)

        plsc.parallel_loop(0, BLK // 128, 1, unroll=2)(grp_body)
        out_cp[p] = pltpu.async_copy(
            buf_i, mkey_hbm.at[pl.ds(base + b * BLK, BLK)], sems_o[p])

    out_cp[0].wait()
    out_cp[1].wait()
    _reduce_hist(hist_v, red_v, NBINS)
    pltpu.sync_copy(red_v, hist_hbm.at[pl.ds(wid * NBINS, NBINS)])


def _hist_pass_body(match_shift, bin_shift, bin_mask, nbins,
                    mkey_hbm, pfx_hbm, hist_hbm,
                    buf_i0, buf_i1, pfx_v, hist_v, red_v, sem_i0, sem_i1):
    wid = _wid()
    base = wid * CHUNK
    laneoff = _lane_iota() * NBINS
    ones = jnp.ones((16,), jnp.int32)
    bufs = (buf_i0, buf_i1)
    sems = (sem_i0, sem_i1)

    pltpu.sync_copy(pfx_hbm, pfx_v)
    pfx = pfx_v[...]
    _zero_hist(hist_v)

    in_cp = [None, None]
    in_cp[0] = pltpu.async_copy(
        mkey_hbm.at[pl.ds(base, BLK)], bufs[0], sems[0])

    for b in range(NBLK):
        p = b & 1
        buf_i = bufs[p]
        in_cp[p].wait()
        if b + 1 < NBLK:
            q = (b + 1) & 1
            in_cp[q] = pltpu.async_copy(
                mkey_hbm.at[pl.ds(base + (b + 1) * BLK, BLK)],
                bufs[q], sems[q])

        def vec_body(i):
            vals = []
            matches = []
            for u in range(8):
                v = buf_i[pl.ds(i * 128 + u * 16, 16)]
                vals.append(v)
                matches.append(lax.shift_right_logical(v, match_shift) == pfx)
            anym = matches[0]
            for u in range(1, 8):
                anym = anym | matches[u]

            @pl.when(jnp.any(anym))
            def _():
                for u in range(8):
                    bin_ = lax.shift_right_logical(vals[u], bin_shift) & bin_mask
                    plsc.addupdate_scatter(hist_v, [bin_ + laneoff], ones,
                                           mask=matches[u])

        plsc.parallel_loop(0, BLK // 128, 1, unroll=2)(vec_body)

    _reduce_hist(hist_v, red_v, nbins)
    pltpu.sync_copy(red_v.at[pl.ds(0, nbins)],
                    hist_hbm.at[pl.ds(wid * nbins, nbins)])


def _apply_body(pred_hbm, mkey_hbm, thr_hbm, out_hbm,
                buf_f0, buf_f1, buf_i0, buf_i1, thr_v,
                sem_f0, sem_f1, sem_i0, sem_i1, sem_o0, sem_o1):
    wid = _wid()
    base = wid * CHUNK
    zero = jnp.zeros((16,), jnp.float32)
    bufs_f = (buf_f0, buf_f1)
    bufs_i = (buf_i0, buf_i1)
    sems_f = (sem_f0, sem_f1)
    sems_i = (sem_i0, sem_i1)
    sems_o = (sem_o0, sem_o1)

    pltpu.sync_copy(thr_hbm, thr_v)
    thr = thr_v[...]

    in_f = [None, None]
    in_i = [None, None]
    out_cp = [None, None]
    in_f[0] = pltpu.async_copy(
        pred_hbm.at[pl.ds(base, BLK)], bufs_f[0], sems_f[0])
    in_i[0] = pltpu.async_copy(
        mkey_hbm.at[pl.ds(base, BLK)], bufs_i[0], sems_i[0])

    for b in range(NBLK):
        p = b & 1
        buf_f, buf_i = bufs_f[p], bufs_i[p]
        in_f[p].wait()
        in_i[p].wait()
        if b + 1 < NBLK:
            q = (b + 1) & 1
            if out_cp[q] is not None:
                out_cp[q].wait()
            in_f[q] = pltpu.async_copy(
                pred_hbm.at[pl.ds(base + (b + 1) * BLK, BLK)],
                bufs_f[q], sems_f[q])
            in_i[q] = pltpu.async_copy(
                mkey_hbm.at[pl.ds(base + (b + 1) * BLK, BLK)],
                bufs_i[q], sems_i[q])

        def vec_body(i):
            for u in range(8):
                sl = pl.ds(i * 128 + u * 16, 16)
                mk = buf_i[sl]
                keep = (mk ^ MSB) > thr
                buf_f[sl] = jnp.where(keep, buf_f[sl], zero)

        plsc.parallel_loop(0, BLK // 128, 1, unroll=2)(vec_body)
        out_cp[p] = pltpu.async_copy(
            buf_f, out_hbm.at[pl.ds(base + b * BLK, BLK)], sems_o[p])

    out_cp[0].wait()
    out_cp[1].wait()


_pass1 = pl.kernel(
    _pass1_body,
    out_type=(
        jax.ShapeDtypeStruct((NW * NBINS,), jnp.int32),
        jax.ShapeDtypeStruct((N,), jnp.int32),
    ),
    mesh=_mesh,
    compiler_params=pltpu.CompilerParams(needs_layout_passes=False),
    scratch_types=[
        pltpu.VMEM((BLK,), jnp.float32),
        pltpu.VMEM((BLK,), jnp.float32),
        pltpu.VMEM((BLK,), jnp.int32),
        pltpu.VMEM((BLK,), jnp.int32),
        pltpu.VMEM((16 * NBINS,), jnp.int32),
        pltpu.VMEM((NBINS,), jnp.int32),
        pltpu.SemaphoreType.DMA,
        pltpu.SemaphoreType.DMA,
        pltpu.SemaphoreType.DMA,
        pltpu.SemaphoreType.DMA,
    ],
)

_hist_scratch = [
    pltpu.VMEM((BLK,), jnp.int32),
    pltpu.VMEM((BLK,), jnp.int32),
    pltpu.VMEM((16,), jnp.int32),
    pltpu.VMEM((16 * NBINS,), jnp.int32),
    pltpu.VMEM((NBINS,), jnp.int32),
    pltpu.SemaphoreType.DMA,
    pltpu.SemaphoreType.DMA,
]

_pass2 = pl.kernel(
    functools.partial(_hist_pass_body, 21, 10, np.int32(0x7FF), 2048),
    out_type=jax.ShapeDtypeStruct((NW * 2048,), jnp.int32),
    mesh=_mesh,
    compiler_params=pltpu.CompilerParams(needs_layout_passes=False),
    scratch_types=list(_hist_scratch),
)

_pass3 = pl.kernel(
    functools.partial(_hist_pass_body, 10, 0, np.int32(0x3FF), 1024),
    out_type=jax.ShapeDtypeStruct((NW * 1024,), jnp.int32),
    mesh=_mesh,
    compiler_params=pltpu.CompilerParams(needs_layout_passes=False),
    scratch_types=list(_hist_scratch),
)

_apply = pl.kernel(
    _apply_body,
    out_type=jax.ShapeDtypeStruct((N,), jnp.float32),
    mesh=_mesh,
    compiler_params=pltpu.CompilerParams(needs_layout_passes=False),
    scratch_types=[
        pltpu.VMEM((BLK,), jnp.float32),
        pltpu.VMEM((BLK,), jnp.float32),
        pltpu.VMEM((BLK,), jnp.int32),
        pltpu.VMEM((BLK,), jnp.int32),
        pltpu.VMEM((16,), jnp.int32),
        pltpu.SemaphoreType.DMA,
        pltpu.SemaphoreType.DMA,
        pltpu.SemaphoreType.DMA,
        pltpu.SemaphoreType.DMA,
        pltpu.SemaphoreType.DMA,
        pltpu.SemaphoreType.DMA,
    ],
)


def _pick(hist_flat, nbins, r):
    g = jnp.sum(hist_flat.reshape(NW, nbins), axis=0)
    c = jnp.cumsum(g)
    b = jnp.argmax(c >= r).astype(jnp.int32)
    r_next = r - (c[b] - g[b])
    return b, r_next


def kernel(pred, points_num):
    r = jnp.int32(N) - jnp.asarray(points_num, jnp.int32)

    hist1, mkey = _pass1(pred)
    b1, r2 = _pick(hist1, 2048, r)

    hist2 = _pass2(mkey, jnp.full((16,), b1, jnp.int32))
    b2, r3 = _pick(hist2, 2048, r2)
    pfx2 = (b1 << 11) | b2

    hist3 = _pass3(mkey, jnp.full((16,), pfx2, jnp.int32))
    b3, _ = _pick(hist3, 1024, r3)
    thresh = (pfx2 << 10) | b3

    sthr = thresh ^ MSB
    return _apply(pred, mkey, jnp.full((16,), sthr, jnp.int32))


# speedup vs baseline: 1.8318x; 1.0045x over previous
import functools

import jax
import jax.numpy as jnp
import numpy as np
from jax import lax
from jax.experimental import pallas as pl
from jax.experimental.pallas import tpu as pltpu
from jax.experimental.pallas import tpu_sc as plsc

N = 2097152
NC, NS, L = 2, 16, 16
NW = NC * NS
CHUNK = N // NW
BLK = 8192
NBLK = CHUNK // BLK
NBINS = 2048
MSB = np.int32(-2147483648)

_mesh = plsc.VectorSubcoreMesh(
    core_axis_name="c", subcore_axis_name="s", num_cores=NC, num_subcores=NS
)


def _wid():
    return lax.axis_index("s") * NC + lax.axis_index("c")


def _lane_iota():
    return lax.iota(jnp.int32, 16)


def _zero_hist(hist_v):
    z = jnp.zeros((16,), jnp.int32)

    def body(i, _):
        for u in range(16):
            hist_v[pl.ds(i * 256 + u * 16, 16)] = z
        return 0

    lax.fori_loop(0, NBINS * 16 // 256, body, 0)


def _reduce_hist(hist_v, red_v, nbins):
    def body(j, _):
        acc = hist_v[pl.ds(j * 16, 16)]
        for l in range(1, 16):
            acc = acc + hist_v[pl.ds(l * NBINS + j * 16, 16)]
        red_v[pl.ds(j * 16, 16)] = acc
        return 0

    lax.fori_loop(0, nbins // 16, body, 0)


def _pass1_body(pred_hbm, hist_hbm, mkey_hbm,
                buf_f0, buf_f1, buf_i0, buf_i1, hist_v, red_v,
                sem_f0, sem_f1, sem_o0, sem_o1):
    wid = _wid()
    base = wid * CHUNK
    lane = _lane_iota()
    laneoff = lane * NBINS
    ones = jnp.ones((16,), jnp.int32)
    bufs_f = (buf_f0, buf_f1)
    bufs_i = (buf_i0, buf_i1)
    sems_f = (sem_f0, sem_f1)
    sems_o = (sem_o0, sem_o1)

    _zero_hist(hist_v)

    in_cp = [None, None]
    out_cp = [None, None]
    in_cp[0] = pltpu.async_copy(
        pred_hbm.at[pl.ds(base, BLK)], bufs_f[0], sems_f[0])

    for b in range(NBLK):
        p = b & 1
        buf_f, buf_i = bufs_f[p], bufs_i[p]
        in_cp[p].wait()
        if b + 1 < NBLK:
            q = (b + 1) & 1
            in_cp[q] = pltpu.async_copy(
                pred_hbm.at[pl.ds(base + (b + 1) * BLK, BLK)],
                bufs_f[q], sems_f[q])
        if out_cp[p] is not None:
            out_cp[p].wait()

        def grp_body(i):
            gidx = lane * 8 + i * 128
            vs = [plsc.load_gather(buf_f, [gidx + j]) for j in range(8)]
            m = vs[0]
            amax = jnp.zeros((16,), jnp.int32)
            for j in range(1, 8):
                gt = vs[j] > m
                m = jnp.where(gt, vs[j], m)
                amax = jnp.where(gt, jnp.int32(j), amax)
            for j in range(8):
                s = plsc.bitcast(vs[j], jnp.int32)
                s = jnp.where(s == MSB, jnp.int32(0), s)
                ukey = s ^ ((s >> 31) | MSB)
                mk = jnp.where(amax == j, jnp.int32(-1), ukey)
                plsc.store_scatter(buf_i, [gidx + j], mk)
                bin_ = lax.shift_right_logical(mk, 21)
                plsc.addupdate_scatter(hist_v, [bin_ + laneoff], ones)

        plsc.parallel_loop(0, BLK // 128, 1, unroll=2)(grp_body)
        out_cp[p] = pltpu.async_copy(
            buf_i, mkey_hbm.at[pl.ds(base + b * BLK, BLK)], sems_o[p])

    out_cp[0].wait()
    out_cp[1].wait()
    _reduce_hist(hist_v, red_v, NBINS)
    pltpu.sync_copy(red_v, hist_hbm.at[wid])


def _hist_pass_body(match_shift, bin_shift, bin_mask, nbins,
                    mkey_hbm, pfx_hbm, hist_hbm,
                    buf_i0, buf_i1, pfx_v, hist_v, red_v, sem_i0, sem_i1):
    wid = _wid()
    base = wid * CHUNK
    laneoff = _lane_iota() * NBINS
    ones = jnp.ones((16,), jnp.int32)
    bufs = (buf_i0, buf_i1)
    sems = (sem_i0, sem_i1)

    pltpu.sync_copy(pfx_hbm, pfx_v)
    pfx = pfx_v[...]
    _zero_hist(hist_v)

    in_cp = [None, None]
    in_cp[0] = pltpu.async_copy(
        mkey_hbm.at[pl.ds(base, BLK)], bufs[0], sems[0])

    for b in range(NBLK):
        p = b & 1
        buf_i = bufs[p]
        in_cp[p].wait()
        if b + 1 < NBLK:
            q = (b + 1) & 1
            in_cp[q] = pltpu.async_copy(
                mkey_hbm.at[pl.ds(base + (b + 1) * BLK, BLK)],
                bufs[q], sems[q])

        def vec_body(i):
            vals = []
            matches = []
            for u in range(8):
                v = buf_i[pl.ds(i * 128 + u * 16, 16)]
                vals.append(v)
                matches.append(lax.shift_right_logical(v, match_shift) == pfx)
            anym = matches[0]
            for u in range(1, 8):
                anym = anym | matches[u]

            @pl.when(jnp.any(anym))
            def _():
                for u in range(8):
                    bin_ = lax.shift_right_logical(vals[u], bin_shift) & bin_mask
                    plsc.addupdate_scatter(hist_v, [bin_ + laneoff], ones,
                                           mask=matches[u])

        plsc.parallel_loop(0, BLK // 128, 1, unroll=2)(vec_body)

    _reduce_hist(hist_v, red_v, nbins)
    pltpu.sync_copy(red_v.at[pl.ds(0, nbins)], hist_hbm.at[wid])


def _apply_body(pred_hbm, mkey_hbm, thr_hbm, out_hbm,
                buf_f0, buf_f1, buf_i0, buf_i1, thr_v,
                sem_f0, sem_f1, sem_i0, sem_i1, sem_o0, sem_o1):
    wid = _wid()
    base = wid * CHUNK
    zero = jnp.zeros((16,), jnp.float32)
    bufs_f = (buf_f0, buf_f1)
    bufs_i = (buf_i0, buf_i1)
    sems_f = (sem_f0, sem_f1)
    sems_i = (sem_i0, sem_i1)
    sems_o = (sem_o0, sem_o1)

    pltpu.sync_copy(thr_hbm, thr_v)
    thr = thr_v[...]

    in_f = [None, None]
    in_i = [None, None]
    out_cp = [None, None]
    in_f[0] = pltpu.async_copy(
        pred_hbm.at[pl.ds(base, BLK)], bufs_f[0], sems_f[0])
    in_i[0] = pltpu.async_copy(
        mkey_hbm.at[pl.ds(base, BLK)], bufs_i[0], sems_i[0])

    for b in range(NBLK):
        p = b & 1
        buf_f, buf_i = bufs_f[p], bufs_i[p]
        in_f[p].wait()
        in_i[p].wait()
        if b + 1 < NBLK:
            q = (b + 1) & 1
            if out_cp[q] is not None:
                out_cp[q].wait()
            in_f[q] = pltpu.async_copy(
                pred_hbm.at[pl.ds(base + (b + 1) * BLK, BLK)],
                bufs_f[q], sems_f[q])
            in_i[q] = pltpu.async_copy(
                mkey_hbm.at[pl.ds(base + (b + 1) * BLK, BLK)],
                bufs_i[q], sems_i[q])

        def vec_body(i):
            for u in range(8):
                sl = pl.ds(i * 128 + u * 16, 16)
                mk = buf_i[sl]
                keep = (mk ^ MSB) > thr
                buf_f[sl] = jnp.where(keep, buf_f[sl], zero)

        plsc.parallel_loop(0, BLK // 128, 1, unroll=2)(vec_body)
        out_cp[p] = pltpu.async_copy(
            buf_f, out_hbm.at[pl.ds(base + b * BLK, BLK)], sems_o[p])

    out_cp[0].wait()
    out_cp[1].wait()


_pass1 = pl.kernel(
    _pass1_body,
    out_type=(
        jax.ShapeDtypeStruct((NW, NBINS), jnp.int32),
        jax.ShapeDtypeStruct((N,), jnp.int32),
    ),
    mesh=_mesh,
    compiler_params=pltpu.CompilerParams(needs_layout_passes=False),
    scratch_types=[
        pltpu.VMEM((BLK,), jnp.float32),
        pltpu.VMEM((BLK,), jnp.float32),
        pltpu.VMEM((BLK,), jnp.int32),
        pltpu.VMEM((BLK,), jnp.int32),
        pltpu.VMEM((16 * NBINS,), jnp.int32),
        pltpu.VMEM((NBINS,), jnp.int32),
        pltpu.SemaphoreType.DMA,
        pltpu.SemaphoreType.DMA,
        pltpu.SemaphoreType.DMA,
        pltpu.SemaphoreType.DMA,
    ],
)

_hist_scratch = [
    pltpu.VMEM((BLK,), jnp.int32),
    pltpu.VMEM((BLK,), jnp.int32),
    pltpu.VMEM((16,), jnp.int32),
    pltpu.VMEM((16 * NBINS,), jnp.int32),
    pltpu.VMEM((NBINS,), jnp.int32),
    pltpu.SemaphoreType.DMA,
    pltpu.SemaphoreType.DMA,
]

_pass2 = pl.kernel(
    functools.partial(_hist_pass_body, 21, 10, np.int32(0x7FF), 2048),
    out_type=jax.ShapeDtypeStruct((NW, 2048), jnp.int32),
    mesh=_mesh,
    compiler_params=pltpu.CompilerParams(needs_layout_passes=False),
    scratch_types=list(_hist_scratch),
)

_pass3 = pl.kernel(
    functools.partial(_hist_pass_body, 10, 0, np.int32(0x3FF), 1024),
    out_type=jax.ShapeDtypeStruct((NW, 1024), jnp.int32),
    mesh=_mesh,
    compiler_params=pltpu.CompilerParams(needs_layout_passes=False),
    scratch_types=list(_hist_scratch),
)

_apply = pl.kernel(
    _apply_body,
    out_type=jax.ShapeDtypeStruct((N,), jnp.float32),
    mesh=_mesh,
    compiler_params=pltpu.CompilerParams(needs_layout_passes=False),
    scratch_types=[
        pltpu.VMEM((BLK,), jnp.float32),
        pltpu.VMEM((BLK,), jnp.float32),
        pltpu.VMEM((BLK,), jnp.int32),
        pltpu.VMEM((BLK,), jnp.int32),
        pltpu.VMEM((16,), jnp.int32),
        pltpu.SemaphoreType.DMA,
        pltpu.SemaphoreType.DMA,
        pltpu.SemaphoreType.DMA,
        pltpu.SemaphoreType.DMA,
        pltpu.SemaphoreType.DMA,
        pltpu.SemaphoreType.DMA,
    ],
)


def _pick(hist, nbins, r):
    g = jnp.sum(hist, axis=0)
    c = jnp.cumsum(g)
    mask = c >= r
    b = jnp.argmax(mask).astype(jnp.int32)
    below = jnp.max(jnp.where(mask, 0, c))
    r_next = r - below
    return b, r_next


def kernel(pred, points_num):
    r = jnp.int32(N) - jnp.asarray(points_num, jnp.int32)

    hist1, mkey = _pass1(pred)
    b1, r2 = _pick(hist1, 2048, r)

    hist2 = _pass2(mkey, jnp.full((16,), b1, jnp.int32))
    b2, r3 = _pick(hist2, 2048, r2)
    pfx2 = (b1 << 11) | b2

    hist3 = _pass3(mkey, jnp.full((16,), pfx2, jnp.int32))
    b3, _ = _pick(hist3, 1024, r3)
    thresh = (pfx2 << 10) | b3

    sthr = thresh ^ MSB
    return _apply(pred, mkey, jnp.full((16,), sthr, jnp.int32))


# speedup vs baseline: 1.9220x; 1.0492x over previous
import functools

import jax
import jax.numpy as jnp
import numpy as np
from jax import lax
from jax.experimental import pallas as pl
from jax.experimental.pallas import tpu as pltpu
from jax.experimental.pallas import tpu_sc as plsc

N = 2097152
NC, NS, L = 2, 16, 16
NW = NC * NS
CHUNK = N // NW
BLK = 16384
NBLK = CHUNK // BLK
NBINS = 2048
MSB = np.int32(-2147483648)

_mesh = plsc.VectorSubcoreMesh(
    core_axis_name="c", subcore_axis_name="s", num_cores=NC, num_subcores=NS
)


def _wid():
    return lax.axis_index("s") * NC + lax.axis_index("c")


def _lane_iota():
    return lax.iota(jnp.int32, 16)


def _zero_hist(hist_v):
    z = jnp.zeros((16,), jnp.int32)

    def body(i, _):
        for u in range(16):
            hist_v[pl.ds(i * 256 + u * 16, 16)] = z
        return 0

    lax.fori_loop(0, NBINS * 16 // 256, body, 0)


def _reduce_hist(hist_v, red_v, nbins):
    def body(j, _):
        acc = hist_v[pl.ds(j * 16, 16)]
        for l in range(1, 16):
            acc = acc + hist_v[pl.ds(l * NBINS + j * 16, 16)]
        red_v[pl.ds(j * 16, 16)] = acc
        return 0

    lax.fori_loop(0, nbins // 16, body, 0)


def _pass1_body(pred_hbm, hist_hbm, mkey_hbm,
                buf_f0, buf_f1, buf_i0, buf_i1, hist_v, red_v,
                sem_f0, sem_f1, sem_o0, sem_o1):
    wid = _wid()
    base = wid * CHUNK
    lane = _lane_iota()
    laneoff = lane * NBINS
    ones = jnp.ones((16,), jnp.int32)
    bufs_f = (buf_f0, buf_f1)
    bufs_i = (buf_i0, buf_i1)
    sems_f = (sem_f0, sem_f1)
    sems_o = (sem_o0, sem_o1)

    _zero_hist(hist_v)

    in_cp = [None, None]
    out_cp = [None, None]
    in_cp[0] = pltpu.async_copy(
        pred_hbm.at[pl.ds(base, BLK)], bufs_f[0], sems_f[0])

    for b in range(NBLK):
        p = b & 1
        buf_f, buf_i = bufs_f[p], bufs_i[p]
        in_cp[p].wait()
        if b + 1 < NBLK:
            q = (b + 1) & 1
            in_cp[q] = pltpu.async_copy(
                pred_hbm.at[pl.ds(base + (b + 1) * BLK, BLK)],
                bufs_f[q], sems_f[q])
        if out_cp[p] is not None:
            out_cp[p].wait()

        def grp_body(i):
            gidx = lane * 8 + i * 128
            vs = [plsc.load_gather(buf_f, [gidx + j]) for j in range(8)]
            m = vs[0]
            amax = jnp.zeros((16,), jnp.int32)
            for j in range(1, 8):
                gt = vs[j] > m
                m = jnp.where(gt, vs[j], m)
                amax = jnp.where(gt, jnp.int32(j), amax)
            for j in range(8):
                s = plsc.bitcast(vs[j], jnp.int32)
                s = jnp.where(s == MSB, jnp.int32(0), s)
                ukey = s ^ ((s >> 31) | MSB)
                mk = jnp.where(amax == j, jnp.int32(-1), ukey)
                plsc.store_scatter(buf_i, [gidx + j], mk)
                bin_ = lax.shift_right_logical(mk, 21)
                plsc.addupdate_scatter(hist_v, [bin_ + laneoff], ones)

        plsc.parallel_loop(0, BLK // 128, 1, unroll=2)(grp_body)
        out_cp[p] = pltpu.async_copy(
            buf_i, mkey_hbm.at[pl.ds(base + b * BLK, BLK)], sems_o[p])

    out_cp[0].wait()
    out_cp[1].wait()
    _reduce_hist(hist_v, red_v, NBINS)
    pltpu.sync_copy(red_v, hist_hbm.at[wid])


def _hist_pass_body(match_shift, bin_shift, bin_mask, nbins,
                    mkey_hbm, pfx_hbm, hist_hbm,
                    buf_i0, buf_i1, pfx_v, hist_v, red_v, sem_i0, sem_i1):
    wid = _wid()
    base = wid * CHUNK
    laneoff = _lane_iota() * NBINS
    ones = jnp.ones((16,), jnp.int32)
    bufs = (buf_i0, buf_i1)
    sems = (sem_i0, sem_i1)

    pltpu.sync_copy(pfx_hbm, pfx_v)
    pfx = pfx_v[...]
    _zero_hist(hist_v)

    in_cp = [None, None]
    in_cp[0] = pltpu.async_copy(
        mkey_hbm.at[pl.ds(base, BLK)], bufs[0], sems[0])

    for b in range(NBLK):
        p = b & 1
        buf_i = bufs[p]
        in_cp[p].wait()
        if b + 1 < NBLK:
            q = (b + 1) & 1
            in_cp[q] = pltpu.async_copy(
                mkey_hbm.at[pl.ds(base + (b + 1) * BLK, BLK)],
                bufs[q], sems[q])

        def vec_body(i):
            vals = []
            matches = []
            for u in range(8):
                v = buf_i[pl.ds(i * 128 + u * 16, 16)]
                vals.append(v)
                matches.append(lax.shift_right_logical(v, match_shift) == pfx)
            anym = matches[0]
            for u in range(1, 8):
                anym = anym | matches[u]

            @pl.when(jnp.any(anym))
            def _():
                for u in range(8):
                    bin_ = lax.shift_right_logical(vals[u], bin_shift) & bin_mask
                    plsc.addupdate_scatter(hist_v, [bin_ + laneoff], ones,
                                           mask=matches[u])

        plsc.parallel_loop(0, BLK // 128, 1, unroll=2)(vec_body)

    _reduce_hist(hist_v, red_v, nbins)
    pltpu.sync_copy(red_v.at[pl.ds(0, nbins)], hist_hbm.at[wid])


def _apply_body(pred_hbm, mkey_hbm, thr_hbm, out_hbm,
                buf_f0, buf_f1, buf_i0, buf_i1, thr_v,
                sem_f0, sem_f1, sem_i0, sem_i1, sem_o0, sem_o1):
    wid = _wid()
    base = wid * CHUNK
    zero = jnp.zeros((16,), jnp.float32)
    bufs_f = (buf_f0, buf_f1)
    bufs_i = (buf_i0, buf_i1)
    sems_f = (sem_f0, sem_f1)
    sems_i = (sem_i0, sem_i1)
    sems_o = (sem_o0, sem_o1)

    pltpu.sync_copy(thr_hbm, thr_v)
    thr = thr_v[...]

    in_f = [None, None]
    in_i = [None, None]
    out_cp = [None, None]
    in_f[0] = pltpu.async_copy(
        pred_hbm.at[pl.ds(base, BLK)], bufs_f[0], sems_f[0])
    in_i[0] = pltpu.async_copy(
        mkey_hbm.at[pl.ds(base, BLK)], bufs_i[0], sems_i[0])

    for b in range(NBLK):
        p = b & 1
        buf_f, buf_i = bufs_f[p], bufs_i[p]
        in_f[p].wait()
        in_i[p].wait()
        if b + 1 < NBLK:
            q = (b + 1) & 1
            if out_cp[q] is not None:
                out_cp[q].wait()
            in_f[q] = pltpu.async_copy(
                pred_hbm.at[pl.ds(base + (b + 1) * BLK, BLK)],
                bufs_f[q], sems_f[q])
            in_i[q] = pltpu.async_copy(
                mkey_hbm.at[pl.ds(base + (b + 1) * BLK, BLK)],
                bufs_i[q], sems_i[q])

        def vec_body(i):
            for u in range(8):
                sl = pl.ds(i * 128 + u * 16, 16)
                mk = buf_i[sl]
                keep = (mk ^ MSB) > thr
                buf_f[sl] = jnp.where(keep, buf_f[sl], zero)

        plsc.parallel_loop(0, BLK // 128, 1, unroll=2)(vec_body)
        out_cp[p] = pltpu.async_copy(
            buf_f, out_hbm.at[pl.ds(base + b * BLK, BLK)], sems_o[p])

    out_cp[0].wait()
    out_cp[1].wait()


_pass1 = pl.kernel(
    _pass1_body,
    out_type=(
        jax.ShapeDtypeStruct((NW, NBINS), jnp.int32),
        jax.ShapeDtypeStruct((N,), jnp.int32),
    ),
    mesh=_mesh,
    compiler_params=pltpu.CompilerParams(needs_layout_passes=False),
    scratch_types=[
        pltpu.VMEM((BLK,), jnp.float32),
        pltpu.VMEM((BLK,), jnp.float32),
        pltpu.VMEM((BLK,), jnp.int32),
        pltpu.VMEM((BLK,), jnp.int32),
        pltpu.VMEM((16 * NBINS,), jnp.int32),
        pltpu.VMEM((NBINS,), jnp.int32),
        pltpu.SemaphoreType.DMA,
        pltpu.SemaphoreType.DMA,
        pltpu.SemaphoreType.DMA,
        pltpu.SemaphoreType.DMA,
    ],
)

_hist_scratch = [
    pltpu.VMEM((BLK,), jnp.int32),
    pltpu.VMEM((BLK,), jnp.int32),
    pltpu.VMEM((16,), jnp.int32),
    pltpu.VMEM((16 * NBINS,), jnp.int32),
    pltpu.VMEM((NBINS,), jnp.int32),
    pltpu.SemaphoreType.DMA,
    pltpu.SemaphoreType.DMA,
]

_pass2 = pl.kernel(
    functools.partial(_hist_pass_body, 21, 10, np.int32(0x7FF), 2048),
    out_type=jax.ShapeDtypeStruct((NW, 2048), jnp.int32),
    mesh=_mesh,
    compiler_params=pltpu.CompilerParams(needs_layout_passes=False),
    scratch_types=list(_hist_scratch),
)

_pass3 = pl.kernel(
    functools.partial(_hist_pass_body, 10, 0, np.int32(0x3FF), 1024),
    out_type=jax.ShapeDtypeStruct((NW, 1024), jnp.int32),
    mesh=_mesh,
    compiler_params=pltpu.CompilerParams(needs_layout_passes=False),
    scratch_types=list(_hist_scratch),
)

_apply = pl.kernel(
    _apply_body,
    out_type=jax.ShapeDtypeStruct((N,), jnp.float32),
    mesh=_mesh,
    compiler_params=pltpu.CompilerParams(needs_layout_passes=False),
    scratch_types=[
        pltpu.VMEM((BLK,), jnp.float32),
        pltpu.VMEM((BLK,), jnp.float32),
        pltpu.VMEM((BLK,), jnp.int32),
        pltpu.VMEM((BLK,), jnp.int32),
        pltpu.VMEM((16,), jnp.int32),
        pltpu.SemaphoreType.DMA,
        pltpu.SemaphoreType.DMA,
        pltpu.SemaphoreType.DMA,
        pltpu.SemaphoreType.DMA,
        pltpu.SemaphoreType.DMA,
        pltpu.SemaphoreType.DMA,
    ],
)


def _pick(hist, nbins, r):
    g = jnp.sum(hist, axis=0)
    c = jnp.cumsum(g)
    mask = c >= r
    b = jnp.argmax(mask).astype(jnp.int32)
    below = jnp.max(jnp.where(mask, 0, c))
    r_next = r - below
    return b, r_next


def kernel(pred, points_num):
    r = jnp.int32(N) - jnp.asarray(points_num, jnp.int32)

    hist1, mkey = _pass1(pred)
    b1, r2 = _pick(hist1, 2048, r)

    hist2 = _pass2(mkey, jnp.full((16,), b1, jnp.int32))
    b2, r3 = _pick(hist2, 2048, r2)
    pfx2 = (b1 << 11) | b2

    hist3 = _pass3(mkey, jnp.full((16,), pfx2, jnp.int32))
    b3, _ = _pick(hist3, 1024, r3)
    thresh = (pfx2 << 10) | b3

    sthr = thresh ^ MSB
    return _apply(pred, mkey, jnp.full((16,), sthr, jnp.int32))
